# initial kernel scaffold (unmeasured)
import jax
import jax.numpy as jnp
from jax import lax
from jax.experimental import pallas as pl
from jax.experimental.pallas import tpu as pltpu


def kernel(
    x,
):
    def body(*refs):
        pass

    out_shape = jax.ShapeDtypeStruct(..., jnp.float32)
    return pl.pallas_call(body, out_shape=out_shape)(...)


# baseline (device time: 30839 ns/iter reference)
import jax
import jax.numpy as jnp
from jax import lax
from jax.experimental import pallas as pl
from jax.experimental.pallas import tpu as pltpu

N_Z = 2


def kernel(x):
    m, n = x.shape

    def body(x_ref, out_ref, send_buf, recv_buf, send_sem, recv_sem):
        my_x = lax.axis_index("x")
        my_y = lax.axis_index("y")
        my_z = lax.axis_index("z")
        partner = (my_x, my_y, 1 - my_z)

        barrier_sem = pltpu.get_barrier_semaphore()
        pl.semaphore_signal(
            barrier_sem, inc=1,
            device_id=partner, device_id_type=pl.DeviceIdType.MESH,
        )
        pl.semaphore_wait(barrier_sem, 1)

        send_buf[...] = x_ref[...].astype(jnp.bfloat16)
        rdma = pltpu.make_async_remote_copy(
            src_ref=send_buf,
            dst_ref=recv_buf,
            send_sem=send_sem,
            recv_sem=recv_sem,
            device_id=partner,
            device_id_type=pl.DeviceIdType.MESH,
        )
        rdma.start()

        out_ref[pl.ds(my_z * m, m), :] = send_buf[...]

        rdma.wait()
        out_ref[pl.ds((1 - my_z) * m, m), :] = recv_buf[...]

    return pl.pallas_call(
        body,
        out_shape=jax.ShapeDtypeStruct((N_Z * m, n), jnp.bfloat16),
        in_specs=[pl.BlockSpec(memory_space=pltpu.VMEM)],
        out_specs=pl.BlockSpec(memory_space=pltpu.VMEM),
        scratch_shapes=[
            pltpu.VMEM((m, n), jnp.bfloat16),
            pltpu.VMEM((m, n), jnp.bfloat16),
            pltpu.SemaphoreType.DMA,
            pltpu.SemaphoreType.DMA,
        ],
        compiler_params=pltpu.CompilerParams(collective_id=0),
    )(x)


# device time: 30687 ns/iter; 1.0050x vs baseline; 1.0050x over previous
import jax
import jax.numpy as jnp
from jax import lax
from jax.experimental import pallas as pl
from jax.experimental.pallas import tpu as pltpu

N_Z = 2


def kernel(x):
    m, n = x.shape

    n_flows = 2
    rows = m // n_flows

    def body(x_ref, out_ref, send_buf, send_sems, recv_sems):
        my_x = lax.axis_index("x")
        my_y = lax.axis_index("y")
        my_z = lax.axis_index("z")
        partner = (my_x, my_y, 1 - my_z)

        barrier_sem = pltpu.get_barrier_semaphore()
        pl.semaphore_signal(
            barrier_sem, inc=1,
            device_id=partner, device_id_type=pl.DeviceIdType.MESH,
        )
        pl.semaphore_wait(barrier_sem, 1)

        send_buf[...] = x_ref[...].astype(jnp.bfloat16)
        rdmas = []
        for f in range(n_flows):
            rdma = pltpu.make_async_remote_copy(
                src_ref=send_buf.at[pl.ds(f * rows, rows)],
                dst_ref=out_ref.at[pl.ds(my_z * m + f * rows, rows)],
                send_sem=send_sems.at[f],
                recv_sem=recv_sems.at[f],
                device_id=partner,
                device_id_type=pl.DeviceIdType.MESH,
            )
            rdma.start()
            rdmas.append(rdma)

        out_ref[pl.ds(my_z * m, m), :] = send_buf[...]

        for rdma in rdmas:
            rdma.wait()

    return pl.pallas_call(
        body,
        out_shape=jax.ShapeDtypeStruct((N_Z * m, n), jnp.bfloat16),
        in_specs=[pl.BlockSpec(memory_space=pltpu.VMEM)],
        out_specs=pl.BlockSpec(memory_space=pltpu.VMEM),
        scratch_shapes=[
            pltpu.VMEM((m, n), jnp.bfloat16),
            pltpu.SemaphoreType.DMA((n_flows,)),
            pltpu.SemaphoreType.DMA((n_flows,)),
        ],
        compiler_params=pltpu.CompilerParams(collective_id=0),
    )(x)


# device time: 28555 ns/iter; 1.0800x vs baseline; 1.0747x over previous
import jax
import jax.numpy as jnp
from jax import lax
from jax.experimental import pallas as pl
from jax.experimental.pallas import tpu as pltpu

N_Z = 2


def kernel(x):
    m, n = x.shape
    half = m // 2
    quarter = m // 4

    def body(x_ref, out_ref, sems_send, sems_recv):
        my_x = lax.axis_index("x")
        my_y = lax.axis_index("y")
        my_z = lax.axis_index("z")
        my_h = (my_x + my_y) % 2
        z_partner = (my_x, my_y, 1 - my_z)
        x_partner = (1 - my_x, my_y, my_z)
        y_partner = (my_x, 1 - my_y, my_z)

        barrier_sem = pltpu.get_barrier_semaphore()
        for nbr in (z_partner, x_partner, y_partner):
            pl.semaphore_signal(
                barrier_sem, inc=1,
                device_id=nbr, device_id_type=pl.DeviceIdType.MESH,
            )
        pl.semaphore_wait(barrier_sem, 3)

        out_ref[pl.ds(my_z * m, m), :] = x_ref[...].astype(jnp.bfloat16)

        z_off = my_z * m + my_h * half
        z_rdma = pltpu.make_async_remote_copy(
            src_ref=out_ref.at[pl.ds(z_off, half)],
            dst_ref=out_ref.at[pl.ds(z_off, half)],
            send_sem=sems_send.at[0],
            recv_sem=sems_recv.at[0],
            device_id=z_partner,
            device_id_type=pl.DeviceIdType.MESH,
        )
        z_rdma.start()
        z_rdma.wait()

        fx_off = (1 - my_z) * m + (2 * my_h) * quarter
        fy_off = (1 - my_z) * m + (2 * my_h + 1) * quarter
        x_rdma = pltpu.make_async_remote_copy(
            src_ref=out_ref.at[pl.ds(fx_off, quarter)],
            dst_ref=out_ref.at[pl.ds(fx_off, quarter)],
            send_sem=sems_send.at[1],
            recv_sem=sems_recv.at[1],
            device_id=x_partner,
            device_id_type=pl.DeviceIdType.MESH,
        )
        y_rdma = pltpu.make_async_remote_copy(
            src_ref=out_ref.at[pl.ds(fy_off, quarter)],
            dst_ref=out_ref.at[pl.ds(fy_off, quarter)],
            send_sem=sems_send.at[2],
            recv_sem=sems_recv.at[2],
            device_id=y_partner,
            device_id_type=pl.DeviceIdType.MESH,
        )
        x_rdma.start()
        y_rdma.start()
        x_rdma.wait()
        y_rdma.wait()

    return pl.pallas_call(
        body,
        out_shape=jax.ShapeDtypeStruct((N_Z * m, n), jnp.bfloat16),
        in_specs=[pl.BlockSpec(memory_space=pltpu.VMEM)],
        out_specs=pl.BlockSpec(memory_space=pltpu.VMEM),
        scratch_shapes=[
            pltpu.SemaphoreType.DMA((3,)),
            pltpu.SemaphoreType.DMA((3,)),
        ],
        compiler_params=pltpu.CompilerParams(collective_id=0),
    )(x)


# device time: 23144 ns/iter; 1.3325x vs baseline; 1.2338x over previous
import jax
import jax.numpy as jnp
from jax import lax
from jax.experimental import pallas as pl
from jax.experimental.pallas import tpu as pltpu

N_Z = 2
N_SUB = 4


def kernel(x):
    m, n = x.shape
    half = m // 2
    quarter = m // 4
    sub = quarter // N_SUB

    def body(x_ref, out_ref, z_send, z_recv, f_send, f_recv):
        my_x = lax.axis_index("x")
        my_y = lax.axis_index("y")
        my_z = lax.axis_index("z")
        my_h = (my_x + my_y) % 2
        z_partner = (my_x, my_y, 1 - my_z)
        x_partner = (1 - my_x, my_y, my_z)
        y_partner = (my_x, 1 - my_y, my_z)

        barrier_sem = pltpu.get_barrier_semaphore()
        for nbr in (z_partner, x_partner, y_partner):
            pl.semaphore_signal(
                barrier_sem, inc=1,
                device_id=nbr, device_id_type=pl.DeviceIdType.MESH,
            )
        pl.semaphore_wait(barrier_sem, 3)

        own = my_z * m
        rcv = (1 - my_z) * m

        z_rdmas = []
        for q in range(2):
            q_off = my_h * half + q * quarter
            out_ref[pl.ds(own + q_off, quarter), :] = (
                x_ref[pl.ds(q_off, quarter), :].astype(jnp.bfloat16)
            )
            for k in range(N_SUB):
                i = q * N_SUB + k
                s_off = own + q_off + k * sub
                rdma = pltpu.make_async_remote_copy(
                    src_ref=out_ref.at[pl.ds(s_off, sub)],
                    dst_ref=out_ref.at[pl.ds(s_off, sub)],
                    send_sem=z_send.at[i],
                    recv_sem=z_recv.at[i],
                    device_id=z_partner,
                    device_id_type=pl.DeviceIdType.MESH,
                )
                rdma.start()
                z_rdmas.append(rdma)

        oh_off = (1 - my_h) * half
        out_ref[pl.ds(own + oh_off, half), :] = (
            x_ref[pl.ds(oh_off, half), :].astype(jnp.bfloat16)
        )

        fwd_rdmas = []
        for q, partner in ((0, x_partner), (1, y_partner)):
            for k in range(N_SUB):
                i = q * N_SUB + k
                z_rdmas[i].wait_recv()
                f_off = rcv + my_h * half + q * quarter + k * sub
                rdma = pltpu.make_async_remote_copy(
                    src_ref=out_ref.at[pl.ds(f_off, sub)],
                    dst_ref=out_ref.at[pl.ds(f_off, sub)],
                    send_sem=f_send.at[i],
                    recv_sem=f_recv.at[i],
                    device_id=partner,
                    device_id_type=pl.DeviceIdType.MESH,
                )
                rdma.start()
                fwd_rdmas.append(rdma)

        for q, partner in ((0, x_partner), (1, y_partner)):
            for k in range(N_SUB):
                i = q * N_SUB + k
                r_off = rcv + (1 - my_h) * half + q * quarter + k * sub
                recv_only = pltpu.make_async_remote_copy(
                    src_ref=out_ref.at[pl.ds(r_off, sub)],
                    dst_ref=out_ref.at[pl.ds(r_off, sub)],
                    send_sem=f_send.at[i],
                    recv_sem=f_recv.at[i],
                    device_id=partner,
                    device_id_type=pl.DeviceIdType.MESH,
                )
                recv_only.wait_recv()

        for rdma in z_rdmas:
            rdma.wait_send()
        for rdma in fwd_rdmas:
            rdma.wait_send()

    n_chunks = 2 * N_SUB
    return pl.pallas_call(
        body,
        out_shape=jax.ShapeDtypeStruct((N_Z * m, n), jnp.bfloat16),
        in_specs=[pl.BlockSpec(memory_space=pltpu.VMEM)],
        out_specs=pl.BlockSpec(memory_space=pltpu.VMEM),
        scratch_shapes=[
            pltpu.SemaphoreType.DMA((n_chunks,)),
            pltpu.SemaphoreType.DMA((n_chunks,)),
            pltpu.SemaphoreType.DMA((n_chunks,)),
            pltpu.SemaphoreType.DMA((n_chunks,)),
        ],
        compiler_params=pltpu.CompilerParams(collective_id=0),
    )(x)


# device time: 21628 ns/iter; 1.4259x vs baseline; 1.0701x over previous
import jax
import jax.numpy as jnp
from jax import lax
from jax.experimental import pallas as pl
from jax.experimental.pallas import tpu as pltpu

N_Z = 2
C = 4


def kernel(x):
    m, n = x.shape
    quarter = m // 4
    sub = quarter // C

    def body(x_ref, out_ref, z_s, z_r, xd_s, xd_r, yd_s, yd_r,
             xr_s, xr_r, yr_s, yr_r):
        my_x = lax.axis_index("x")
        my_y = lax.axis_index("y")
        my_z = lax.axis_index("z")
        z_partner = (my_x, my_y, 1 - my_z)
        x_partner = (1 - my_x, my_y, my_z)
        y_partner = (my_x, 1 - my_y, my_z)

        q_me = (2 * my_x + my_y) * quarter
        q_xp = (2 * (1 - my_x) + my_y) * quarter
        q_yp = (2 * my_x + (1 - my_y)) * quarter
        q_dg = (2 * (1 - my_x) + (1 - my_y)) * quarter

        barrier_sem = pltpu.get_barrier_semaphore()
        for nbr in (z_partner, x_partner, y_partner):
            pl.semaphore_signal(
                barrier_sem, inc=1,
                device_id=nbr, device_id_type=pl.DeviceIdType.MESH,
            )
        pl.semaphore_wait(barrier_sem, 3)

        own = my_z * m
        rcv = (1 - my_z) * m

        out_ref[pl.ds(own + q_me, quarter), :] = (
            x_ref[pl.ds(q_me, quarter), :].astype(jnp.bfloat16)
        )
        z_rdmas = []
        for c in range(C):
            off = own + q_me + c * sub
            rdma = pltpu.make_async_remote_copy(
                src_ref=out_ref.at[pl.ds(off, sub)],
                dst_ref=out_ref.at[pl.ds(off, sub)],
                send_sem=z_s.at[c],
                recv_sem=z_r.at[c],
                device_id=z_partner,
                device_id_type=pl.DeviceIdType.MESH,
            )
            rdma.start()
            z_rdmas.append(rdma)

        for j in range(1, 4):
            q_off = (q_me + j * quarter) % m
            out_ref[pl.ds(own + q_off, quarter), :] = (
                x_ref[pl.ds(q_off, quarter), :].astype(jnp.bfloat16)
            )

        direct_rdmas = []
        for c in range(C):
            z_rdmas[c].wait_recv()
            off = rcv + q_me + c * sub
            for sems, partner in ((( xd_s, xd_r), x_partner),
                                  (((yd_s, yd_r)), y_partner)):
                s, r = sems
                rdma = pltpu.make_async_remote_copy(
                    src_ref=out_ref.at[pl.ds(off, sub)],
                    dst_ref=out_ref.at[pl.ds(off, sub)],
                    send_sem=s.at[c],
                    recv_sem=r.at[c],
                    device_id=partner,
                    device_id_type=pl.DeviceIdType.MESH,
                )
                rdma.start()
                direct_rdmas.append(rdma)

        def recv_only(off, sem):
            return pltpu.make_async_remote_copy(
                src_ref=out_ref.at[pl.ds(off, sub)],
                dst_ref=out_ref.at[pl.ds(off, sub)],
                send_sem=z_s.at[0],
                recv_sem=sem,
                device_id=z_partner,
                device_id_type=pl.DeviceIdType.MESH,
            )

        relay_rdmas = []
        for c in range(C // 2):
            recv_only(rcv + q_yp + c * sub, yd_r.at[c]).wait_recv()
            off = rcv + q_yp + c * sub
            rdma = pltpu.make_async_remote_copy(
                src_ref=out_ref.at[pl.ds(off, sub)],
                dst_ref=out_ref.at[pl.ds(off, sub)],
                send_sem=xr_s.at[c],
                recv_sem=xr_r.at[c],
                device_id=x_partner,
                device_id_type=pl.DeviceIdType.MESH,
            )
            rdma.start()
            relay_rdmas.append(rdma)
        for c in range(C // 2):
            recv_only(rcv + q_xp + (C // 2 + c) * sub,
                      xd_r.at[C // 2 + c]).wait_recv()
            off = rcv + q_xp + (C // 2 + c) * sub
            rdma = pltpu.make_async_remote_copy(
                src_ref=out_ref.at[pl.ds(off, sub)],
                dst_ref=out_ref.at[pl.ds(off, sub)],
                send_sem=yr_s.at[c],
                recv_sem=yr_r.at[c],
                device_id=y_partner,
                device_id_type=pl.DeviceIdType.MESH,
            )
            rdma.start()
            relay_rdmas.append(rdma)

        for c in range(C // 2):
            recv_only(rcv + q_xp + c * sub, xd_r.at[c]).wait_recv()
            recv_only(rcv + q_yp + (C // 2 + c) * sub,
                      yd_r.at[C // 2 + c]).wait_recv()
            recv_only(rcv + q_dg + c * sub, xr_r.at[c]).wait_recv()
            recv_only(rcv + q_dg + (C // 2 + c) * sub,
                      yr_r.at[c]).wait_recv()

        for rdma in z_rdmas:
            rdma.wait_send()
        for rdma in direct_rdmas:
            rdma.wait_send()
        for rdma in relay_rdmas:
            rdma.wait_send()

    return pl.pallas_call(
        body,
        out_shape=jax.ShapeDtypeStruct((N_Z * m, n), jnp.bfloat16),
        in_specs=[pl.BlockSpec(memory_space=pltpu.VMEM)],
        out_specs=pl.BlockSpec(memory_space=pltpu.VMEM),
        scratch_shapes=[
            pltpu.SemaphoreType.DMA((C,)),
            pltpu.SemaphoreType.DMA((C,)),
            pltpu.SemaphoreType.DMA((C,)),
            pltpu.SemaphoreType.DMA((C,)),
            pltpu.SemaphoreType.DMA((C,)),
            pltpu.SemaphoreType.DMA((C,)),
            pltpu.SemaphoreType.DMA((C // 2,)),
            pltpu.SemaphoreType.DMA((C // 2,)),
            pltpu.SemaphoreType.DMA((C // 2,)),
            pltpu.SemaphoreType.DMA((C // 2,)),
        ],
        compiler_params=pltpu.CompilerParams(collective_id=0),
    )(x)
